# baseline (speedup 1.0000x reference)
import jax
import jax.numpy as jnp
from jax import lax
from jax.experimental import pallas as pl
from jax.experimental.pallas import tpu as pltpu
from jax.experimental.pallas import tpu_sc as plsc

B, N, D, M = 4, 4096, 1024, 1024
NC, NS = 2, 16
NW = NC * NS
RW = M // NW
T = 32
DC = D // 16


def _zero_acc(acc_v):
    def zr(r, _):
        def zc(jc, _):
            acc_v[r, pl.ds(jc * 16, 16)] = jnp.zeros((16,), jnp.float32)
            return 0

        lax.fori_loop(0, DC, zc, 0, unroll=8)
        return 0

    lax.fori_loop(0, RW, zr, 0)


def _sc_body(emb_hbm, ids_hbm, out_hbm, ids_v, acc_v, rb0, bnd_v, scale_v):
    c = lax.axis_index("c")
    s = lax.axis_index("s")
    w = s * NC + c
    m0 = w * RW

    _zero_acc(acc_v)

    for b in range(B):
        pltpu.sync_copy(ids_hbm.at[b], ids_v.at[pl.ds(0, N)])

        for g in range(3):
            tgt = jax.lax.broadcasted_iota(jnp.int32, (16,), 0) + (
                m0 + g * 16)
            lo0 = jnp.zeros((16,), jnp.int32)
            hi0 = jnp.full((16,), N, jnp.int32)

            def sbody(it, carry):
                lo, hi = carry
                mid = (lo + hi) >> 1
                midc = jnp.minimum(mid, N - 1)
                v = plsc.load_gather(ids_v, [midc])
                less = v < tgt
                return (jnp.where(less, mid + 1, lo),
                        jnp.where(less, hi, mid))

            lo, _hi = lax.fori_loop(0, 12, sbody, (lo0, hi0))
            bnd_v[pl.ds(g * 16, 16)] = lo

        for g in range(2):
            cnt = (bnd_v[pl.ds(g * 16 + 1, 16)] -
                   bnd_v[pl.ds(g * 16, 16)]).astype(jnp.float32)
            scale_v[pl.ds(g * 16, 16)] = 1.0 / (cnt + 1.0)

        tst = bnd_v[pl.ds(0, 16)][0]
        ten = bnd_v[pl.ds(RW, 16)][0]
        abase = (tst // 8) * 8
        nch = (ten - abase + T - 1) // T

        def ch_body(ci, _):
            base0 = abase + ci * T
            base = pl.multiple_of(jnp.minimum(base0, N - T), 8)
            pltpu.sync_copy(emb_hbm.at[b, pl.ds(base, T)], rb0)
            lo_t = jnp.maximum(base0, tst)

            def row_body(r, _):
                t = base + r
                valid = jnp.logical_and(t >= lo_t, t < ten)

                @pl.when(valid)
                def _():
                    mt = ids_v[pl.ds(t, 16)][0] - m0

                    def cb(jc, _):
                        sl = pl.ds(jc * 16, 16)
                        plsc.addupdate(acc_v.at[mt, sl], rb0[r, sl])
                        return 0

                    lax.fori_loop(0, DC, cb, 0, unroll=8)

                return 0

            lax.fori_loop(0, T, row_body, 0)
            return 0

        lax.fori_loop(0, nch, ch_body, 0)

        def fr(r, _):
            sc = plsc.load_gather(scale_v, [jnp.full((16,), r, jnp.int32)])

            def fc(jc, _):
                sl = pl.ds(jc * 16, 16)
                acc_v[r, sl] = acc_v[r, sl] * sc
                return 0

            lax.fori_loop(0, DC, fc, 0, unroll=8)
            return 0

        lax.fori_loop(0, RW, fr, 0)
        pltpu.sync_copy(acc_v, out_hbm.at[b, pl.ds(pl.multiple_of(m0, 8), RW)])
        _zero_acc(acc_v)


@jax.jit
def _sc_pool(embeddings, position_ids):
    mesh = plsc.VectorSubcoreMesh(
        core_axis_name="c", subcore_axis_name="s",
        num_cores=NC, num_subcores=NS)
    return pl.kernel(
        _sc_body,
        out_type=jax.ShapeDtypeStruct((B, M, D), jnp.float32),
        mesh=mesh,
        compiler_params=pltpu.CompilerParams(needs_layout_passes=False),
        scratch_types=[
            pltpu.VMEM((N + 16,), jnp.int32),
            pltpu.VMEM((RW, D), jnp.float32),
            pltpu.VMEM((T, D), jnp.float32),
            pltpu.VMEM((48,), jnp.int32),
            pltpu.VMEM((RW,), jnp.float32),
        ],
    )(embeddings, position_ids)


def kernel(embeddings, position_ids):
    return _sc_pool(embeddings, position_ids)
